# baseline (speedup 1.0000x reference)
import functools

import jax
import jax.numpy as jnp
from jax import lax
from jax.experimental import pallas as pl
from jax.experimental.pallas import tpu as pltpu
from jax.experimental.pallas import tpu_sc as plsc

NUM_RUNS = 4
P_DROP = 0.1
EB = 128
NSC = 2
NTEC = 16
RB = 400

_f32 = jnp.float32


NSLOT = 4
CH = 4096


def _make_segment_sum(RN, W, RE_pad, dump_rows):
    HALF = RN // NSC
    ZROWS = (HALF + dump_rows + NTEC * 8 - 1) // (NTEC * 8) * 8
    ACC = ZROWS * NTEC
    OUTR = HALF // NTEC // 8 * 8
    REM = HALF - OUTR * NTEC
    stripe = RE_pad // NTEC
    NCHUNK = stripe // CH
    NBC = CH // EB

    mesh = plsc.VectorSubcoreMesh(
        core_axis_name="c", subcore_axis_name="s",
        num_cores=NSC, num_subcores=NTEC)

    def body(rsrc, rdst, zrows, h0, h1, h2, h3,
             o0, o1, o2, o3, sidx, didx, rows, acc, *sems):
        gsems = sems[:NSLOT]
        ssems = sems[NSLOT:]
        c = lax.axis_index("c")
        s = lax.axis_index("s")
        base_row = c * HALF
        hs = [h0, h1, h2, h3]
        os_ = [o0, o1, o2, o3]
        zdummy = zrows.at[pl.ds(0, EB)]
        for f in range(4):
            pltpu.sync_copy(zrows, acc.at[pl.ds(s * ZROWS, ZROWS)])
            plsc.subcore_barrier()

            def chunk(ci, carry):
                crow = (s * stripe + ci * CH) // EB
                pltpu.sync_copy(rsrc.at[pl.ds(crow, NBC)], sidx)
                pltpu.sync_copy(rdst.at[pl.ds(crow, NBC)], didx)
                for sl in range(NSLOT):
                    pltpu.async_copy(hs[f].at[sidx.at[sl]], rows.at[sl],
                                     gsems[sl])

                def step(p, carry2):
                    for sl in range(NSLOT):
                        b = NSLOT * p + sl
                        drow = didx.at[b]
                        for j in range(EB // 16):
                            v = drow[pl.ds(j * 16, 16)]
                            rel = v - base_row
                            ok = (rel >= 0) & (rel < HALF)
                            drow[pl.ds(j * 16, 16)] = jnp.where(ok, rel, HALF)
                        pltpu.make_async_copy(zdummy, rows.at[sl],
                                              gsems[sl]).wait()
                        pltpu.async_copy(rows.at[sl], acc.at[drow],
                                         ssems[sl], add=True)
                        pltpu.make_async_copy(zdummy, rows.at[sl],
                                              ssems[sl]).wait()

                        @pl.when(b + NSLOT < NBC)
                        def _():
                            pltpu.async_copy(hs[f].at[sidx.at[b + NSLOT]],
                                             rows.at[sl], gsems[sl])
                    return carry2

                lax.fori_loop(0, NBC // NSLOT, step, 0, unroll=False)
                return carry

            lax.fori_loop(0, NCHUNK, chunk, 0, unroll=False)
            plsc.subcore_barrier()
            pltpu.sync_copy(
                acc.at[pl.ds(s * OUTR, OUTR)],
                os_[f].at[pl.ds(base_row + s * OUTR, OUTR)])
            if REM:
                @pl.when(s == 0)
                def _():
                    pltpu.sync_copy(
                        acc.at[pl.ds(NTEC * OUTR, REM)],
                        os_[f].at[pl.ds(base_row + NTEC * OUTR, REM)])
            plsc.subcore_barrier()

    out = [jax.ShapeDtypeStruct((RN, W), _f32)] * 4
    return pl.kernel(
        body, out_type=out, mesh=mesh,
        compiler_params=pltpu.CompilerParams(use_tc_tiling_on_sc=False),
        scratch_types=[
            pltpu.VMEM((NBC, EB), jnp.int32),
            pltpu.VMEM((NBC, EB), jnp.int32),
            pltpu.VMEM((NSLOT, EB, W), _f32),
            pltpu.VMEM_SHARED((ACC, W), _f32),
        ] + [pltpu.SemaphoreType.DMA] * (2 * NSLOT),
        name=f"gin_segsum_w{W}")


def _drop_expand(x, keep, W):
    n, fin = x.shape
    nb = n // RB

    def body(x_ref, k_ref, *outs):
        xb = x_ref[...] * k_ref[0]
        for j in range(4):
            outs[j][...] = xb[:, j * W:(j + 1) * W]

    grid = (NUM_RUNS, nb)
    return pl.pallas_call(
        body,
        grid=grid,
        in_specs=[
            pl.BlockSpec((RB, fin), lambda r, i: (i, 0)),
            pl.BlockSpec((1, RB, 1), lambda r, i: (r, i, 0)),
        ],
        out_specs=[pl.BlockSpec((RB, W), lambda r, i: (r * (n // RB) + i, 0))
                   for _ in range(4)],
        out_shape=[jax.ShapeDtypeStruct((NUM_RUNS * n, W), _f32)] * 4,
        name="drop_expand",
    )(x, keep)


def _mm1_stats(hs, aggs, w1, b1):
    RN = hs[0].shape[0]
    W = hs[0].shape[1]
    fin = 4 * W
    dim = w1.shape[1]
    nb = RN // RB

    def body(h0, h1, h2, h3, a0, a1, a2, a3, w_ref, b_ref, y_ref, s_ref, q_ref):
        i = pl.program_id(0)
        hh = jnp.concatenate([h0[...], h1[...], h2[...], h3[...]], axis=1)
        aa = jnp.concatenate([a0[...], a1[...], a2[...], a3[...]], axis=1)
        y = lax.dot_general((hh + aa), w_ref[...], (((1,), (0,)), ((), ())),
                            preferred_element_type=_f32,
                            precision=lax.Precision.HIGHEST) + b_ref[...]
        y_ref[...] = y
        ps = y.reshape(RB // 8, 8, dim).sum(axis=0)
        pq = (y * y).reshape(RB // 8, 8, dim).sum(axis=0)

        @pl.when(i == 0)
        def _():
            s_ref[...] = ps
            q_ref[...] = pq

        @pl.when(i > 0)
        def _():
            s_ref[...] += ps
            q_ref[...] += pq

    cspec = [pl.BlockSpec((RB, W), lambda i: (i, 0)) for _ in range(8)]
    return pl.pallas_call(
        body,
        grid=(nb,),
        in_specs=cspec + [
            pl.BlockSpec((fin, dim), lambda i: (0, 0)),
            pl.BlockSpec((1, dim), lambda i: (0, 0)),
        ],
        out_specs=[
            pl.BlockSpec((RB, dim), lambda i: (i, 0)),
            pl.BlockSpec((8, dim), lambda i: (0, 0)),
            pl.BlockSpec((8, dim), lambda i: (0, 0)),
        ],
        out_shape=[
            jax.ShapeDtypeStruct((RN, dim), _f32),
            jax.ShapeDtypeStruct((8, dim), _f32),
            jax.ShapeDtypeStruct((8, dim), _f32),
        ],
        name="gin_mm1",
    )(*hs, *aggs, w1, b1)


def _bn_relu_mm2(y1, s1, q1, g1, bb1, w2, b2):
    RN, dim = y1.shape
    nb = RN // RB
    inv_n = 1.0 / RN

    def body(y_ref, s_ref, q_ref, g_ref, bb_ref, w_ref, b_ref,
             o_ref, so_ref, qo_ref):
        i = pl.program_id(0)
        mu = s_ref[...].sum(axis=0, keepdims=True) * inv_n
        var = q_ref[...].sum(axis=0, keepdims=True) * inv_n - mu * mu
        scale = g_ref[...] * lax.rsqrt(var + 1e-5)
        a = jnp.maximum((y_ref[...] - mu) * scale + bb_ref[...], 0.0)
        y = lax.dot_general(a, w_ref[...], (((1,), (0,)), ((), ())),
                            preferred_element_type=_f32,
                            precision=lax.Precision.HIGHEST) + b_ref[...]
        o_ref[...] = y
        ps = y.reshape(RB // 8, 8, dim).sum(axis=0)
        pq = (y * y).reshape(RB // 8, 8, dim).sum(axis=0)

        @pl.when(i == 0)
        def _():
            so_ref[...] = ps
            qo_ref[...] = pq

        @pl.when(i > 0)
        def _():
            so_ref[...] += ps
            qo_ref[...] += pq

    return pl.pallas_call(
        body,
        grid=(nb,),
        in_specs=[
            pl.BlockSpec((RB, dim), lambda i: (i, 0)),
            pl.BlockSpec((8, dim), lambda i: (0, 0)),
            pl.BlockSpec((8, dim), lambda i: (0, 0)),
            pl.BlockSpec((1, dim), lambda i: (0, 0)),
            pl.BlockSpec((1, dim), lambda i: (0, 0)),
            pl.BlockSpec((dim, dim), lambda i: (0, 0)),
            pl.BlockSpec((1, dim), lambda i: (0, 0)),
        ],
        out_specs=[
            pl.BlockSpec((RB, dim), lambda i: (i, 0)),
            pl.BlockSpec((8, dim), lambda i: (0, 0)),
            pl.BlockSpec((8, dim), lambda i: (0, 0)),
        ],
        out_shape=[
            jax.ShapeDtypeStruct((RN, dim), _f32),
            jax.ShapeDtypeStruct((8, dim), _f32),
            jax.ShapeDtypeStruct((8, dim), _f32),
        ],
        name="gin_mm2",
    )(y1, s1, q1, g1, bb1, w2, b2)


def _bn_relu_mean(y2, s2, q2, g2, bb2, n):
    RN, dim = y2.shape
    W = dim // 4
    nb = n // RB
    inv_n = 1.0 / RN
    inv_r = 1.0 / NUM_RUNS

    def body(y_ref, s_ref, q_ref, g_ref, bb_ref, h0, h1, h2, h3, m_ref):
        r = pl.program_id(1)
        mu = s_ref[...].sum(axis=0, keepdims=True) * inv_n
        var = q_ref[...].sum(axis=0, keepdims=True) * inv_n - mu * mu
        scale = g_ref[...] * lax.rsqrt(var + 1e-5)
        h = jnp.maximum((y_ref[...] - mu) * scale + bb_ref[...], 0.0)
        outs = [h0, h1, h2, h3]
        for j in range(4):
            outs[j][...] = h[:, j * W:(j + 1) * W]

        @pl.when(r == 0)
        def _():
            m_ref[...] = h * inv_r

        @pl.when(r > 0)
        def _():
            m_ref[...] += h * inv_r

    return pl.pallas_call(
        body,
        grid=(nb, NUM_RUNS),
        in_specs=[
            pl.BlockSpec((RB, dim), lambda i, r: (r * (RN // NUM_RUNS // RB) + i, 0)),
            pl.BlockSpec((8, dim), lambda i, r: (0, 0)),
            pl.BlockSpec((8, dim), lambda i, r: (0, 0)),
            pl.BlockSpec((1, dim), lambda i, r: (0, 0)),
            pl.BlockSpec((1, dim), lambda i, r: (0, 0)),
        ],
        out_specs=[pl.BlockSpec((RB, W),
                                lambda i, r: (r * (RN // NUM_RUNS // RB) + i, 0))
                   for _ in range(4)] +
                  [pl.BlockSpec((RB, dim), lambda i, r: (i, 0))],
        out_shape=[jax.ShapeDtypeStruct((RN, W), _f32)] * 4 +
                  [jax.ShapeDtypeStruct((n, dim), _f32)],
        name="gin_bn_mean",
    )(y2, s2, q2, g2, bb2)


def _readout(x, kf, ms, wcat, bsum):
    n, fin = x.shape
    nb = n // RB
    c = wcat.shape[1]
    kdim = wcat.shape[0]

    def body(x_ref, kf_ref, m1, m2, m3, m4, w_ref, b_ref, o_ref):
        m0 = x_ref[...] * kf_ref[...]
        mcat = jnp.concatenate(
            [m0, m1[...], m2[...], m3[...], m4[...]], axis=1)
        logits = lax.dot_general(mcat, w_ref[...], (((1,), (0,)), ((), ())),
                                 preferred_element_type=_f32,
                            precision=lax.Precision.HIGHEST) + b_ref[...]
        mx = jnp.max(logits, axis=1, keepdims=True)
        sh = logits - mx
        lse = jnp.log(jnp.sum(jnp.exp(sh), axis=1, keepdims=True))
        o_ref[...] = sh - lse

    dim = ms[0].shape[1]
    return pl.pallas_call(
        body,
        grid=(nb,),
        in_specs=[
            pl.BlockSpec((RB, fin), lambda i: (i, 0)),
            pl.BlockSpec((RB, 1), lambda i: (i, 0)),
        ] + [pl.BlockSpec((RB, dim), lambda i: (i, 0)) for _ in range(4)] + [
            pl.BlockSpec((kdim, c), lambda i: (0, 0)),
            pl.BlockSpec((1, c), lambda i: (0, 0)),
        ],
        out_specs=pl.BlockSpec((RB, c), lambda i: (i, 0)),
        out_shape=jax.ShapeDtypeStruct((n, c), _f32),
        name="gin_readout",
    )(x, kf, *ms, wcat, bsum)


def kernel(x, edge_index, batch, params):
    convs, bns, fcs = params
    n, fin = x.shape
    R = NUM_RUNS
    RN = R * n
    num_layers = len(convs)

    drop = jax.random.bernoulli(jax.random.key(42), P_DROP, (R, n))
    keep = (1.0 - drop.astype(_f32)).reshape(R, n, 1)
    kf = keep.mean(axis=0)

    src = edge_index[0]
    dst = edge_index[1]
    offset = jnp.max(edge_index) + 1
    run_off = (jnp.arange(R, dtype=edge_index.dtype)[:, None] * offset)
    rsrc = (src[None, :] + run_off).reshape(-1)
    rdst = (dst[None, :] + run_off).reshape(-1)
    RE = rsrc.shape[0]
    RE_pad = ((RE + NTEC * CH - 1) // (NTEC * CH)) * (NTEC * CH)
    if RE_pad != RE:
        pad = RE_pad - RE
        rsrc = jnp.concatenate([rsrc, jnp.zeros((pad,), rsrc.dtype)])
        rdst = jnp.concatenate([rdst, jnp.full((pad,), RN, rdst.dtype)])
    rsrc = rsrc.reshape(RE_pad // EB, EB)
    rdst = rdst.reshape(RE_pad // EB, EB)

    dump_rows = 96
    nz = (RN // NSC + dump_rows + NTEC * 8 - 1) // (NTEC * 8) * 8
    zrows32 = jnp.zeros((nz, fin // 4), _f32)
    zrows64 = None

    hs = _drop_expand(x, keep, fin // 4)

    ms = []
    for i in range(num_layers):
        w1, b1, g1, bb1, w2, b2 = convs[i]
        g, b = bns[i]
        W = hs[0].shape[1]
        if W == fin // 4:
            zr = zrows32
        else:
            if zrows64 is None:
                zrows64 = jnp.zeros((nz, W), _f32)
            zr = zrows64
        seg = _make_segment_sum(RN, W, RE_pad, dump_rows)
        aggs = seg(rsrc, rdst, zr, *hs)
        y1, s1, q1 = _mm1_stats(hs, aggs, w1, b1.reshape(1, -1))
        y2, s2, q2 = _bn_relu_mm2(y1, s1, q1, g1.reshape(1, -1),
                                  bb1.reshape(1, -1), w2, b2.reshape(1, -1))
        *hs, m = _bn_relu_mean(y2, s2, q2, g.reshape(1, -1),
                               b.reshape(1, -1), n)
        ms.append(m)

    wcat = jnp.concatenate([w for (w, _) in fcs], axis=0)
    bsum = sum(bb for (_, bb) in fcs).reshape(1, -1)
    return _readout(x, kf, ms, wcat, bsum)


# speedup vs baseline: 1.3942x; 1.3942x over previous
import functools

import jax
import jax.numpy as jnp
from jax import lax
from jax.experimental import pallas as pl
from jax.experimental.pallas import tpu as pltpu
from jax.experimental.pallas import tpu_sc as plsc

NUM_RUNS = 4
P_DROP = 0.1
EB = 128
NSC = 2
NTEC = 16
RB = 400

_f32 = jnp.float32


NSLOT = 4
CH = 4096
CHB = 8192
CB = CHB + EB
NPROD = NSC * NTEC


def _bucket_cap(RE_pad):
    stripe32 = RE_pad // NPROD
    nchb = stripe32 // CHB
    return (stripe32 + 128 * nchb + CB + 4095) // 4096 * 4096


def _make_bucket(RN, RE_pad):
    HALF = RN // NSC
    stripe32 = RE_pad // NPROD
    NCHB = stripe32 // CHB
    cap = _bucket_cap(RE_pad)

    mesh = plsc.VectorSubcoreMesh(
        core_axis_name="c", subcore_axis_name="s",
        num_cores=NSC, num_subcores=NTEC)

    def body(rsrc, rdst, dsrc, ddst, bsrc, bdst, cnts,
             sch, dch, bs0, bd0, bs1, bd1, cbuf):
        c = lax.axis_index("c")
        s = lax.axis_index("s")
        w = c * NTEC + s
        base = w * stripe32
        zero16 = jnp.zeros((16,), jnp.int32)
        dump16 = jnp.full((16,), RN, jnp.int32)
        iota16 = lax.iota(jnp.int32, 16)
        r0 = jnp.int32(0)
        r1 = jnp.int32(0)
        for ci in range(NCHB):
            pltpu.sync_copy(rsrc.at[pl.ds(base + ci * CHB, CHB)], sch)
            pltpu.sync_copy(rdst.at[pl.ds(base + ci * CHB, CHB)], dch)

            def grp(g, carry):
                c0, c1 = carry
                s16 = sch[pl.ds(g * 16, 16)]
                d16 = dch[pl.ds(g * 16, 16)]
                m0 = d16 < HALF
                m0i = jnp.where(m0, 1, 0).astype(jnp.int32)
                excl0 = jnp.zeros((16,), jnp.int32)
                for j in range(15):
                    excl0 = excl0 + jnp.where(iota16 > j, m0i[j], 0)
                excl1 = iota16 - excl0
                trash = CB + iota16
                pos0 = jnp.where(m0, c0 + excl0, trash)
                pos1 = jnp.where(m0, trash, c1 + excl1)
                plsc.store_scatter(bs0, [pos0], s16)
                plsc.store_scatter(bd0, [pos0], d16)
                plsc.store_scatter(bs1, [pos1], s16)
                plsc.store_scatter(bd1, [pos1], d16)
                k0 = plsc.all_reduce_population_count(m0)[0]
                return c0 + k0, c1 + (16 - k0)

            c0, c1 = lax.fori_loop(0, CHB // 16, grp,
                                   (jnp.int32(0), jnp.int32(0)),
                                   unroll=False)
            for j in range(EB // 16):
                bs0[pl.ds(c0 + 16 * j, 16)] = zero16
                bd0[pl.ds(c0 + 16 * j, 16)] = dump16
                bs1[pl.ds(c1 + 16 * j, 16)] = zero16
                bd1[pl.ds(c1 + 16 * j, 16)] = dump16
            pltpu.sync_copy(bs0.at[pl.ds(0, CB)], bsrc.at[0, w, pl.ds(r0, CB)])
            pltpu.sync_copy(bd0.at[pl.ds(0, CB)], bdst.at[0, w, pl.ds(r0, CB)])
            pltpu.sync_copy(bs1.at[pl.ds(0, CB)], bsrc.at[1, w, pl.ds(r1, CB)])
            pltpu.sync_copy(bd1.at[pl.ds(0, CB)], bdst.at[1, w, pl.ds(r1, CB)])
            r0 = r0 + (c0 + 127) // 128 * 128
            r1 = r1 + (c1 + 127) // 128 * 128
        pltpu.sync_copy(dsrc, bsrc.at[0, w, pl.ds(r0, CB)])
        pltpu.sync_copy(ddst, bdst.at[0, w, pl.ds(r0, CB)])
        pltpu.sync_copy(dsrc, bsrc.at[1, w, pl.ds(r1, CB)])
        pltpu.sync_copy(ddst, bdst.at[1, w, pl.ds(r1, CB)])
        n0 = (r0 + 4095) // 4096 * 4096
        n1 = (r1 + 4095) // 4096 * 4096
        cbuf[pl.ds(0, 16)] = jnp.broadcast_to(n0, (16,))
        pltpu.sync_copy(cbuf, cnts.at[0, w])
        cbuf[pl.ds(0, 16)] = jnp.broadcast_to(n1, (16,))
        pltpu.sync_copy(cbuf, cnts.at[1, w])

    out = [
        jax.ShapeDtypeStruct((NSC, NPROD, cap), jnp.int32),
        jax.ShapeDtypeStruct((NSC, NPROD, cap), jnp.int32),
        jax.ShapeDtypeStruct((NSC, NPROD, 16), jnp.int32),
    ]
    return pl.kernel(
        body, out_type=out, mesh=mesh,
        compiler_params=pltpu.CompilerParams(use_tc_tiling_on_sc=False,
                                             needs_layout_passes=False),
        scratch_types=[
            pltpu.VMEM((CHB,), jnp.int32),
            pltpu.VMEM((CHB,), jnp.int32),
            pltpu.VMEM((CB + 16,), jnp.int32),
            pltpu.VMEM((CB + 16,), jnp.int32),
            pltpu.VMEM((CB + 16,), jnp.int32),
            pltpu.VMEM((CB + 16,), jnp.int32),
            pltpu.VMEM((16,), jnp.int32),
        ],
        name="gin_bucket")


def _make_segment_sum(RN, W, RE_pad, dump_rows):
    HALF = RN // NSC
    ZROWS = (HALF + dump_rows + NTEC * 8 - 1) // (NTEC * 8) * 8
    ACC = ZROWS * NTEC
    OUTR = HALF // NTEC // 8 * 8
    REM = HALF - OUTR * NTEC
    NBC = CH // EB

    mesh = plsc.VectorSubcoreMesh(
        core_axis_name="c", subcore_axis_name="s",
        num_cores=NSC, num_subcores=NTEC)

    def body(bsrc, bdst, cnts, zrows, h0, h1, h2, h3,
             o0, o1, o2, o3, sidx, didx, rows, cvec, acc, *sems):
        gsems = sems[:NSLOT]
        ssems = sems[NSLOT:]
        c = lax.axis_index("c")
        s = lax.axis_index("s")
        base_row = c * HALF
        hs = [h0, h1, h2, h3]
        os_ = [o0, o1, o2, o3]
        iota16 = lax.iota(jnp.int32, 16)
        zdummy = zrows.at[pl.ds(0, EB)]
        prods = [s * 2, s * 2 + 1]
        ncs = []
        for reg in range(2):
            pltpu.sync_copy(cnts.at[c, prods[reg]], cvec)
            cnt = cvec[pl.ds(0, 16)][0]
            ncs.append(cnt // CH)
        for f in range(4):
            pltpu.sync_copy(zrows, acc.at[pl.ds(s * ZROWS, ZROWS)])
            plsc.subcore_barrier()

            for reg in range(2):
                w = prods[reg]

                def chunk(ci, carry, w=w):
                    crow = ci * NBC
                    pltpu.sync_copy(bsrc.at[c, w, pl.ds(crow, NBC)], sidx)
                    pltpu.sync_copy(bdst.at[c, w, pl.ds(crow, NBC)], didx)
                    for sl in range(NSLOT):
                        pltpu.async_copy(hs[f].at[sidx.at[sl]], rows.at[sl],
                                         gsems[sl])

                    def step(p, carry2):
                        for sl in range(NSLOT):
                            b = NSLOT * p + sl
                            drow = didx.at[b]
                            for j in range(EB // 16):
                                v = drow[pl.ds(j * 16, 16)]
                                rel = v - base_row
                                ok = (rel >= 0) & (rel < HALF)
                                drow[pl.ds(j * 16, 16)] = jnp.where(
                                    ok, rel, HALF)
                            pltpu.make_async_copy(zdummy, rows.at[sl],
                                                  gsems[sl]).wait()
                            pltpu.async_copy(rows.at[sl], acc.at[drow],
                                             ssems[sl], add=True)
                            pltpu.make_async_copy(zdummy, rows.at[sl],
                                                  ssems[sl]).wait()

                            @pl.when(b + NSLOT < NBC)
                            def _():
                                pltpu.async_copy(
                                    hs[f].at[sidx.at[b + NSLOT]],
                                    rows.at[sl], gsems[sl])
                        return carry2

                    lax.fori_loop(0, NBC // NSLOT, step, 0, unroll=False)
                    return carry

                lax.fori_loop(0, ncs[reg], chunk, 0, unroll=False)
            plsc.subcore_barrier()
            pltpu.sync_copy(
                acc.at[pl.ds(s * OUTR, OUTR)],
                os_[f].at[pl.ds(base_row + s * OUTR, OUTR)])
            if REM:
                @pl.when(s == 0)
                def _():
                    pltpu.sync_copy(
                        acc.at[pl.ds(NTEC * OUTR, REM)],
                        os_[f].at[pl.ds(base_row + NTEC * OUTR, REM)])
            plsc.subcore_barrier()

    out = [jax.ShapeDtypeStruct((RN, W), _f32)] * 4
    return pl.kernel(
        body, out_type=out, mesh=mesh,
        compiler_params=pltpu.CompilerParams(use_tc_tiling_on_sc=False),
        scratch_types=[
            pltpu.VMEM((NBC, EB), jnp.int32),
            pltpu.VMEM((NBC, EB), jnp.int32),
            pltpu.VMEM((NSLOT, EB, W), _f32),
            pltpu.VMEM((16,), jnp.int32),
            pltpu.VMEM_SHARED((ACC, W), _f32),
        ] + [pltpu.SemaphoreType.DMA] * (2 * NSLOT),
        name=f"gin_segsum_w{W}")


def _drop_expand(x, keep, W):
    n, fin = x.shape
    nb = n // RB

    def body(x_ref, k_ref, *outs):
        xb = x_ref[...] * k_ref[0]
        for j in range(4):
            outs[j][...] = xb[:, j * W:(j + 1) * W]

    grid = (NUM_RUNS, nb)
    return pl.pallas_call(
        body,
        grid=grid,
        in_specs=[
            pl.BlockSpec((RB, fin), lambda r, i: (i, 0)),
            pl.BlockSpec((1, RB, 1), lambda r, i: (r, i, 0)),
        ],
        out_specs=[pl.BlockSpec((RB, W), lambda r, i: (r * (n // RB) + i, 0))
                   for _ in range(4)],
        out_shape=[jax.ShapeDtypeStruct((NUM_RUNS * n, W), _f32)] * 4,
        name="drop_expand",
    )(x, keep)


def _mm1_stats(hs, aggs, w1, b1):
    RN = hs[0].shape[0]
    W = hs[0].shape[1]
    fin = 4 * W
    dim = w1.shape[1]
    nb = RN // RB

    def body(h0, h1, h2, h3, a0, a1, a2, a3, w_ref, b_ref, y_ref, s_ref, q_ref):
        i = pl.program_id(0)
        hh = jnp.concatenate([h0[...], h1[...], h2[...], h3[...]], axis=1)
        aa = jnp.concatenate([a0[...], a1[...], a2[...], a3[...]], axis=1)
        y = lax.dot_general((hh + aa), w_ref[...], (((1,), (0,)), ((), ())),
                            preferred_element_type=_f32,
                            precision=lax.Precision.HIGHEST) + b_ref[...]
        y_ref[...] = y
        ps = y.reshape(RB // 8, 8, dim).sum(axis=0)
        pq = (y * y).reshape(RB // 8, 8, dim).sum(axis=0)

        @pl.when(i == 0)
        def _():
            s_ref[...] = ps
            q_ref[...] = pq

        @pl.when(i > 0)
        def _():
            s_ref[...] += ps
            q_ref[...] += pq

    cspec = [pl.BlockSpec((RB, W), lambda i: (i, 0)) for _ in range(8)]
    return pl.pallas_call(
        body,
        grid=(nb,),
        in_specs=cspec + [
            pl.BlockSpec((fin, dim), lambda i: (0, 0)),
            pl.BlockSpec((1, dim), lambda i: (0, 0)),
        ],
        out_specs=[
            pl.BlockSpec((RB, dim), lambda i: (i, 0)),
            pl.BlockSpec((8, dim), lambda i: (0, 0)),
            pl.BlockSpec((8, dim), lambda i: (0, 0)),
        ],
        out_shape=[
            jax.ShapeDtypeStruct((RN, dim), _f32),
            jax.ShapeDtypeStruct((8, dim), _f32),
            jax.ShapeDtypeStruct((8, dim), _f32),
        ],
        name="gin_mm1",
    )(*hs, *aggs, w1, b1)


def _bn_relu_mm2(y1, s1, q1, g1, bb1, w2, b2):
    RN, dim = y1.shape
    nb = RN // RB
    inv_n = 1.0 / RN

    def body(y_ref, s_ref, q_ref, g_ref, bb_ref, w_ref, b_ref,
             o_ref, so_ref, qo_ref):
        i = pl.program_id(0)
        mu = s_ref[...].sum(axis=0, keepdims=True) * inv_n
        var = q_ref[...].sum(axis=0, keepdims=True) * inv_n - mu * mu
        scale = g_ref[...] * lax.rsqrt(var + 1e-5)
        a = jnp.maximum((y_ref[...] - mu) * scale + bb_ref[...], 0.0)
        y = lax.dot_general(a, w_ref[...], (((1,), (0,)), ((), ())),
                            preferred_element_type=_f32,
                            precision=lax.Precision.HIGHEST) + b_ref[...]
        o_ref[...] = y
        ps = y.reshape(RB // 8, 8, dim).sum(axis=0)
        pq = (y * y).reshape(RB // 8, 8, dim).sum(axis=0)

        @pl.when(i == 0)
        def _():
            so_ref[...] = ps
            qo_ref[...] = pq

        @pl.when(i > 0)
        def _():
            so_ref[...] += ps
            qo_ref[...] += pq

    return pl.pallas_call(
        body,
        grid=(nb,),
        in_specs=[
            pl.BlockSpec((RB, dim), lambda i: (i, 0)),
            pl.BlockSpec((8, dim), lambda i: (0, 0)),
            pl.BlockSpec((8, dim), lambda i: (0, 0)),
            pl.BlockSpec((1, dim), lambda i: (0, 0)),
            pl.BlockSpec((1, dim), lambda i: (0, 0)),
            pl.BlockSpec((dim, dim), lambda i: (0, 0)),
            pl.BlockSpec((1, dim), lambda i: (0, 0)),
        ],
        out_specs=[
            pl.BlockSpec((RB, dim), lambda i: (i, 0)),
            pl.BlockSpec((8, dim), lambda i: (0, 0)),
            pl.BlockSpec((8, dim), lambda i: (0, 0)),
        ],
        out_shape=[
            jax.ShapeDtypeStruct((RN, dim), _f32),
            jax.ShapeDtypeStruct((8, dim), _f32),
            jax.ShapeDtypeStruct((8, dim), _f32),
        ],
        name="gin_mm2",
    )(y1, s1, q1, g1, bb1, w2, b2)


def _bn_relu_mean(y2, s2, q2, g2, bb2, n):
    RN, dim = y2.shape
    W = dim // 4
    nb = n // RB
    inv_n = 1.0 / RN
    inv_r = 1.0 / NUM_RUNS

    def body(y_ref, s_ref, q_ref, g_ref, bb_ref, h0, h1, h2, h3, m_ref):
        r = pl.program_id(1)
        mu = s_ref[...].sum(axis=0, keepdims=True) * inv_n
        var = q_ref[...].sum(axis=0, keepdims=True) * inv_n - mu * mu
        scale = g_ref[...] * lax.rsqrt(var + 1e-5)
        h = jnp.maximum((y_ref[...] - mu) * scale + bb_ref[...], 0.0)
        outs = [h0, h1, h2, h3]
        for j in range(4):
            outs[j][...] = h[:, j * W:(j + 1) * W]

        @pl.when(r == 0)
        def _():
            m_ref[...] = h * inv_r

        @pl.when(r > 0)
        def _():
            m_ref[...] += h * inv_r

    return pl.pallas_call(
        body,
        grid=(nb, NUM_RUNS),
        in_specs=[
            pl.BlockSpec((RB, dim), lambda i, r: (r * (RN // NUM_RUNS // RB) + i, 0)),
            pl.BlockSpec((8, dim), lambda i, r: (0, 0)),
            pl.BlockSpec((8, dim), lambda i, r: (0, 0)),
            pl.BlockSpec((1, dim), lambda i, r: (0, 0)),
            pl.BlockSpec((1, dim), lambda i, r: (0, 0)),
        ],
        out_specs=[pl.BlockSpec((RB, W),
                                lambda i, r: (r * (RN // NUM_RUNS // RB) + i, 0))
                   for _ in range(4)] +
                  [pl.BlockSpec((RB, dim), lambda i, r: (i, 0))],
        out_shape=[jax.ShapeDtypeStruct((RN, W), _f32)] * 4 +
                  [jax.ShapeDtypeStruct((n, dim), _f32)],
        name="gin_bn_mean",
    )(y2, s2, q2, g2, bb2)


def _readout(x, kf, ms, wcat, bsum):
    n, fin = x.shape
    nb = n // RB
    c = wcat.shape[1]
    kdim = wcat.shape[0]

    def body(x_ref, kf_ref, m1, m2, m3, m4, w_ref, b_ref, o_ref):
        m0 = x_ref[...] * kf_ref[...]
        mcat = jnp.concatenate(
            [m0, m1[...], m2[...], m3[...], m4[...]], axis=1)
        logits = lax.dot_general(mcat, w_ref[...], (((1,), (0,)), ((), ())),
                                 preferred_element_type=_f32,
                            precision=lax.Precision.HIGHEST) + b_ref[...]
        mx = jnp.max(logits, axis=1, keepdims=True)
        sh = logits - mx
        lse = jnp.log(jnp.sum(jnp.exp(sh), axis=1, keepdims=True))
        o_ref[...] = sh - lse

    dim = ms[0].shape[1]
    return pl.pallas_call(
        body,
        grid=(nb,),
        in_specs=[
            pl.BlockSpec((RB, fin), lambda i: (i, 0)),
            pl.BlockSpec((RB, 1), lambda i: (i, 0)),
        ] + [pl.BlockSpec((RB, dim), lambda i: (i, 0)) for _ in range(4)] + [
            pl.BlockSpec((kdim, c), lambda i: (0, 0)),
            pl.BlockSpec((1, c), lambda i: (0, 0)),
        ],
        out_specs=pl.BlockSpec((RB, c), lambda i: (i, 0)),
        out_shape=jax.ShapeDtypeStruct((n, c), _f32),
        name="gin_readout",
    )(x, kf, *ms, wcat, bsum)


def kernel(x, edge_index, batch, params):
    convs, bns, fcs = params
    n, fin = x.shape
    R = NUM_RUNS
    RN = R * n
    num_layers = len(convs)

    drop = jax.random.bernoulli(jax.random.key(42), P_DROP, (R, n))
    keep = (1.0 - drop.astype(_f32)).reshape(R, n, 1)
    kf = keep.mean(axis=0)

    src = edge_index[0]
    dst = edge_index[1]
    offset = jnp.max(edge_index) + 1
    run_off = (jnp.arange(R, dtype=edge_index.dtype)[:, None] * offset)
    rsrc = (src[None, :] + run_off).reshape(-1)
    rdst = (dst[None, :] + run_off).reshape(-1)
    RE = rsrc.shape[0]
    gran = NPROD * CHB
    RE_pad = ((RE + gran - 1) // gran) * gran
    if RE_pad != RE:
        pad = RE_pad - RE
        rsrc = jnp.concatenate([rsrc, jnp.zeros((pad,), rsrc.dtype)])
        rdst = jnp.concatenate([rdst, jnp.full((pad,), RN, rdst.dtype)])

    dsrc = jnp.zeros((CB,), jnp.int32)
    ddst = jnp.full((CB,), RN, jnp.int32)
    bsrc, bdst, cnts = _make_bucket(RN, RE_pad)(rsrc, rdst, dsrc, ddst)
    cap = _bucket_cap(RE_pad)
    bsrc = bsrc.reshape(NSC, NPROD, cap // EB, EB)
    bdst = bdst.reshape(NSC, NPROD, cap // EB, EB)

    dump_rows = 96
    nz = (RN // NSC + dump_rows + NTEC * 8 - 1) // (NTEC * 8) * 8
    zrows32 = jnp.zeros((nz, fin // 4), _f32)
    zrows64 = None

    hs = _drop_expand(x, keep, fin // 4)

    ms = []
    for i in range(num_layers):
        w1, b1, g1, bb1, w2, b2 = convs[i]
        g, b = bns[i]
        W = hs[0].shape[1]
        if W == fin // 4:
            zr = zrows32
        else:
            if zrows64 is None:
                zrows64 = jnp.zeros((nz, W), _f32)
            zr = zrows64
        seg = _make_segment_sum(RN, W, RE_pad, dump_rows)
        aggs = seg(bsrc, bdst, cnts, zr, *hs)
        y1, s1, q1 = _mm1_stats(hs, aggs, w1, b1.reshape(1, -1))
        y2, s2, q2 = _bn_relu_mm2(y1, s1, q1, g1.reshape(1, -1),
                                  bb1.reshape(1, -1), w2, b2.reshape(1, -1))
        *hs, m = _bn_relu_mean(y2, s2, q2, g.reshape(1, -1),
                               b.reshape(1, -1), n)
        ms.append(m)

    wcat = jnp.concatenate([w for (w, _) in fcs], axis=0)
    bsum = sum(bb for (_, bb) in fcs).reshape(1, -1)
    return _readout(x, kf, ms, wcat, bsum)


# speedup vs baseline: 1.7888x; 1.2830x over previous
import jax
import jax.numpy as jnp
from jax import lax
from jax.experimental import pallas as pl
from jax.experimental.pallas import tpu as pltpu
from jax.experimental.pallas import tpu_sc as plsc

NUM_RUNS = 4
P_DROP = 0.1
EB = 128
NSC = 2
NTEC = 16
RB = 400
NBKT = 4
NCOL = 2

_f32 = jnp.float32


CH = 4096
CHB = 8192
CB = CHB + EB
NPROD = NSC * NTEC


def _bucket_cap(RE_pad):
    stripe32 = RE_pad // NPROD
    nchb = stripe32 // CHB
    return (stripe32 + 128 * nchb + CB + 4095) // 4096 * 4096


def _make_bucket(RN, RE_pad):
    QUART = RN // NBKT
    stripe32 = RE_pad // NPROD
    NCHB = stripe32 // CHB
    cap = _bucket_cap(RE_pad)

    mesh = plsc.VectorSubcoreMesh(
        core_axis_name="c", subcore_axis_name="s",
        num_cores=NSC, num_subcores=NTEC)

    def body(rsrc, rdst, dsrc, ddst, bsrc, bdst, cnts,
             sch, dch, b0, b1, b2, b3, b4, b5, b6, b7, cbuf):
        c = lax.axis_index("c")
        s = lax.axis_index("s")
        w = c * NTEC + s
        base = w * stripe32
        zero16 = jnp.zeros((16,), jnp.int32)
        dump16 = jnp.full((16,), RN, jnp.int32)
        iota16 = lax.iota(jnp.int32, 16)
        bss = [b0, b2, b4, b6]
        bds = [b1, b3, b5, b7]
        rs = [jnp.int32(0)] * NBKT
        for ci in range(NCHB):
            pltpu.sync_copy(rsrc.at[pl.ds(base + ci * CHB, CHB)], sch)
            pltpu.sync_copy(rdst.at[pl.ds(base + ci * CHB, CHB)], dch)

            def grp(g, carry):
                s16 = sch[pl.ds(g * 16, 16)]
                d16 = dch[pl.ds(g * 16, 16)]
                out = []
                for q in range(NBKT):
                    mq = (d16 >= q * QUART) & (d16 < (q + 1) * QUART)
                    mqi = jnp.where(mq, 1, 0).astype(jnp.int32)
                    excl = jnp.zeros((16,), jnp.int32)
                    for j in range(15):
                        excl = excl + jnp.where(iota16 > j, mqi[j], 0)
                    pos = jnp.where(mq, carry[q] + excl, CB + iota16)
                    plsc.store_scatter(bss[q], [pos], s16)
                    plsc.store_scatter(bds[q], [pos], d16)
                    kq = plsc.all_reduce_population_count(mq)[0]
                    out.append(carry[q] + kq)
                return tuple(out)

            cs = lax.fori_loop(0, CHB // 16, grp,
                               tuple(jnp.int32(0) for _ in range(NBKT)),
                               unroll=False)
            for q in range(NBKT):
                for j in range(EB // 16):
                    bss[q][pl.ds(cs[q] + 16 * j, 16)] = zero16
                    bds[q][pl.ds(cs[q] + 16 * j, 16)] = dump16
                pltpu.sync_copy(bss[q].at[pl.ds(0, CB)],
                                bsrc.at[q, w, pl.ds(rs[q], CB)])
                pltpu.sync_copy(bds[q].at[pl.ds(0, CB)],
                                bdst.at[q, w, pl.ds(rs[q], CB)])
                rs[q] = rs[q] + (cs[q] + 127) // 128 * 128
        for q in range(NBKT):
            pltpu.sync_copy(dsrc, bsrc.at[q, w, pl.ds(rs[q], CB)])
            pltpu.sync_copy(ddst, bdst.at[q, w, pl.ds(rs[q], CB)])
            nq = (rs[q] + 4095) // 4096 * 4096
            cbuf[pl.ds(0, 16)] = jnp.broadcast_to(nq, (16,))
            pltpu.sync_copy(cbuf, cnts.at[q, w])

    out = [
        jax.ShapeDtypeStruct((NBKT, NPROD, cap), jnp.int32),
        jax.ShapeDtypeStruct((NBKT, NPROD, cap), jnp.int32),
        jax.ShapeDtypeStruct((NBKT, NPROD, 16), jnp.int32),
    ]
    return pl.kernel(
        body, out_type=out, mesh=mesh,
        compiler_params=pltpu.CompilerParams(use_tc_tiling_on_sc=False,
                                             needs_layout_passes=False),
        scratch_types=[
            pltpu.VMEM((CHB,), jnp.int32),
            pltpu.VMEM((CHB,), jnp.int32),
        ] + [pltpu.VMEM((CB + 16,), jnp.int32) for _ in range(2 * NBKT)] + [
            pltpu.VMEM((16,), jnp.int32),
        ],
        name="gin_bucket")


def _make_segment_sum(RN, W, dump_rows):
    QUART = RN // NBKT
    ZROWS = (QUART + dump_rows + NTEC * 8 - 1) // (NTEC * 8) * 8
    ACC = ZROWS * NTEC
    OUTR = QUART // NTEC // 8 * 8
    REM = QUART - OUTR * NTEC
    NBC = CH // EB
    NSLOT = 4 if W <= 64 else 2

    mesh = plsc.VectorSubcoreMesh(
        core_axis_name="c", subcore_axis_name="s",
        num_cores=NSC, num_subcores=NTEC)

    def body(bsrc, bdst, cnts, zrows, h0, h1,
             o0, o1, sidx, didx, rows, cvec, acc, *sems):
        gsems = sems[:NSLOT]
        ssems = sems[NSLOT:]
        c = lax.axis_index("c")
        s = lax.axis_index("s")
        hs = [h0, h1]
        os_ = [o0, o1]
        zdummy = zrows.at[pl.ds(0, EB)]
        prods = [s * 2, s * 2 + 1]
        for sub in range(2):
            qq = c * 2 + sub
            base_row = qq * QUART
            ncs = []
            for reg in range(2):
                pltpu.sync_copy(cnts.at[qq, prods[reg]], cvec)
                ncs.append(cvec[pl.ds(0, 16)][0] // CH)
            for f in range(NCOL):
                pltpu.sync_copy(zrows, acc.at[pl.ds(s * ZROWS, ZROWS)])
                plsc.subcore_barrier()

                for reg in range(2):
                    w = prods[reg]

                    def chunk(ci, carry, w=w):
                        crow = ci * NBC
                        pltpu.sync_copy(bsrc.at[qq, w, pl.ds(crow, NBC)],
                                        sidx)
                        pltpu.sync_copy(bdst.at[qq, w, pl.ds(crow, NBC)],
                                        didx)
                        for sl in range(NSLOT):
                            pltpu.async_copy(hs[f].at[sidx.at[sl]],
                                             rows.at[sl], gsems[sl])

                        def step(p, carry2):
                            for sl in range(NSLOT):
                                b = NSLOT * p + sl
                                drow = didx.at[b]
                                for j in range(EB // 16):
                                    v = drow[pl.ds(j * 16, 16)]
                                    rel = v - base_row
                                    ok = (rel >= 0) & (rel < QUART)
                                    drow[pl.ds(j * 16, 16)] = jnp.where(
                                        ok, rel, QUART)
                                pltpu.make_async_copy(zdummy, rows.at[sl],
                                                      gsems[sl]).wait()
                                pltpu.async_copy(rows.at[sl], acc.at[drow],
                                                 ssems[sl], add=True)
                                pltpu.make_async_copy(zdummy, rows.at[sl],
                                                      ssems[sl]).wait()

                                @pl.when(b + NSLOT < NBC)
                                def _():
                                    pltpu.async_copy(
                                        hs[f].at[sidx.at[b + NSLOT]],
                                        rows.at[sl], gsems[sl])
                            return carry2

                        lax.fori_loop(0, NBC // NSLOT, step, 0, unroll=False)
                        return carry

                    lax.fori_loop(0, ncs[reg], chunk, 0, unroll=False)
                plsc.subcore_barrier()
                pltpu.sync_copy(
                    acc.at[pl.ds(s * OUTR, OUTR)],
                    os_[f].at[pl.ds(base_row + s * OUTR, OUTR)])
                if REM:
                    @pl.when(s == 0)
                    def _():
                        pltpu.sync_copy(
                            acc.at[pl.ds(NTEC * OUTR, REM)],
                            os_[f].at[pl.ds(base_row + NTEC * OUTR, REM)])
                plsc.subcore_barrier()

    out = [jax.ShapeDtypeStruct((RN, W), _f32)] * NCOL
    return pl.kernel(
        body, out_type=out, mesh=mesh,
        compiler_params=pltpu.CompilerParams(use_tc_tiling_on_sc=False),
        scratch_types=[
            pltpu.VMEM((NBC, EB), jnp.int32),
            pltpu.VMEM((NBC, EB), jnp.int32),
            pltpu.VMEM((NSLOT, EB, W), _f32),
            pltpu.VMEM((16,), jnp.int32),
            pltpu.VMEM_SHARED((ACC, W), _f32),
        ] + [pltpu.SemaphoreType.DMA] * (2 * NSLOT),
        name=f"gin_segsum_w{W}")


def _drop_expand(x, keep, W):
    n, fin = x.shape
    nb = n // RB

    def body(x_ref, k_ref, *outs):
        xb = x_ref[...] * k_ref[0]
        for j in range(NCOL):
            outs[j][...] = xb[:, j * W:(j + 1) * W]

    return pl.pallas_call(
        body,
        grid=(NUM_RUNS, nb),
        in_specs=[
            pl.BlockSpec((RB, fin), lambda r, i: (i, 0)),
            pl.BlockSpec((1, RB, 1), lambda r, i: (r, i, 0)),
        ],
        out_specs=[pl.BlockSpec((RB, W), lambda r, i: (r * (n // RB) + i, 0))
                   for _ in range(NCOL)],
        out_shape=[jax.ShapeDtypeStruct((NUM_RUNS * n, W), _f32)] * NCOL,
        name="drop_expand",
    )(x, keep)


def _mm1_stats(hs, aggs, w1, b1):
    RN, W = hs[0].shape
    fin = NCOL * W
    dim = w1.shape[1]
    nb = RN // RB

    def body(h0, h1, a0, a1, w_ref, b_ref, y_ref, s_ref, q_ref):
        i = pl.program_id(0)
        hh = jnp.concatenate([h0[...], h1[...]], axis=1)
        aa = jnp.concatenate([a0[...], a1[...]], axis=1)
        y = lax.dot_general((hh + aa), w_ref[...], (((1,), (0,)), ((), ())),
                            preferred_element_type=_f32,
                            precision=lax.Precision.HIGHEST) + b_ref[...]
        y_ref[...] = y
        ps = y.reshape(RB // 8, 8, dim).sum(axis=0)
        pq = (y * y).reshape(RB // 8, 8, dim).sum(axis=0)

        @pl.when(i == 0)
        def _():
            s_ref[...] = ps
            q_ref[...] = pq

        @pl.when(i > 0)
        def _():
            s_ref[...] += ps
            q_ref[...] += pq

    cspec = [pl.BlockSpec((RB, W), lambda i: (i, 0)) for _ in range(4)]
    return pl.pallas_call(
        body,
        grid=(nb,),
        in_specs=cspec + [
            pl.BlockSpec((fin, dim), lambda i: (0, 0)),
            pl.BlockSpec((1, dim), lambda i: (0, 0)),
        ],
        out_specs=[
            pl.BlockSpec((RB, dim), lambda i: (i, 0)),
            pl.BlockSpec((8, dim), lambda i: (0, 0)),
            pl.BlockSpec((8, dim), lambda i: (0, 0)),
        ],
        out_shape=[
            jax.ShapeDtypeStruct((RN, dim), _f32),
            jax.ShapeDtypeStruct((8, dim), _f32),
            jax.ShapeDtypeStruct((8, dim), _f32),
        ],
        name="gin_mm1",
    )(*hs, *aggs, w1, b1)


def _bn_relu_mm2(y1, s1, q1, g1, bb1, w2, b2):
    RN, dim = y1.shape
    nb = RN // RB
    inv_n = 1.0 / RN

    def body(y_ref, s_ref, q_ref, g_ref, bb_ref, w_ref, b_ref,
             o_ref, so_ref, qo_ref):
        i = pl.program_id(0)
        mu = s_ref[...].sum(axis=0, keepdims=True) * inv_n
        var = q_ref[...].sum(axis=0, keepdims=True) * inv_n - mu * mu
        scale = g_ref[...] * lax.rsqrt(var + 1e-5)
        a = jnp.maximum((y_ref[...] - mu) * scale + bb_ref[...], 0.0)
        y = lax.dot_general(a, w_ref[...], (((1,), (0,)), ((), ())),
                            preferred_element_type=_f32,
                            precision=lax.Precision.HIGHEST) + b_ref[...]
        o_ref[...] = y
        ps = y.reshape(RB // 8, 8, dim).sum(axis=0)
        pq = (y * y).reshape(RB // 8, 8, dim).sum(axis=0)

        @pl.when(i == 0)
        def _():
            so_ref[...] = ps
            qo_ref[...] = pq

        @pl.when(i > 0)
        def _():
            so_ref[...] += ps
            qo_ref[...] += pq

    return pl.pallas_call(
        body,
        grid=(nb,),
        in_specs=[
            pl.BlockSpec((RB, dim), lambda i: (i, 0)),
            pl.BlockSpec((8, dim), lambda i: (0, 0)),
            pl.BlockSpec((8, dim), lambda i: (0, 0)),
            pl.BlockSpec((1, dim), lambda i: (0, 0)),
            pl.BlockSpec((1, dim), lambda i: (0, 0)),
            pl.BlockSpec((dim, dim), lambda i: (0, 0)),
            pl.BlockSpec((1, dim), lambda i: (0, 0)),
        ],
        out_specs=[
            pl.BlockSpec((RB, dim), lambda i: (i, 0)),
            pl.BlockSpec((8, dim), lambda i: (0, 0)),
            pl.BlockSpec((8, dim), lambda i: (0, 0)),
        ],
        out_shape=[
            jax.ShapeDtypeStruct((RN, dim), _f32),
            jax.ShapeDtypeStruct((8, dim), _f32),
            jax.ShapeDtypeStruct((8, dim), _f32),
        ],
        name="gin_mm2",
    )(y1, s1, q1, g1, bb1, w2, b2)


def _bn_relu_mean(y2, s2, q2, g2, bb2, n):
    RN, dim = y2.shape
    W = dim // NCOL
    nb = n // RB
    inv_n = 1.0 / RN
    inv_r = 1.0 / NUM_RUNS

    def body(y_ref, s_ref, q_ref, g_ref, bb_ref, h0, h1, m_ref):
        r = pl.program_id(1)
        mu = s_ref[...].sum(axis=0, keepdims=True) * inv_n
        var = q_ref[...].sum(axis=0, keepdims=True) * inv_n - mu * mu
        scale = g_ref[...] * lax.rsqrt(var + 1e-5)
        h = jnp.maximum((y_ref[...] - mu) * scale + bb_ref[...], 0.0)
        outs = [h0, h1]
        for j in range(NCOL):
            outs[j][...] = h[:, j * W:(j + 1) * W]

        @pl.when(r == 0)
        def _():
            m_ref[...] = h * inv_r

        @pl.when(r > 0)
        def _():
            m_ref[...] += h * inv_r

    return pl.pallas_call(
        body,
        grid=(nb, NUM_RUNS),
        in_specs=[
            pl.BlockSpec((RB, dim),
                         lambda i, r: (r * (RN // NUM_RUNS // RB) + i, 0)),
            pl.BlockSpec((8, dim), lambda i, r: (0, 0)),
            pl.BlockSpec((8, dim), lambda i, r: (0, 0)),
            pl.BlockSpec((1, dim), lambda i, r: (0, 0)),
            pl.BlockSpec((1, dim), lambda i, r: (0, 0)),
        ],
        out_specs=[pl.BlockSpec((RB, W),
                                lambda i, r: (r * (RN // NUM_RUNS // RB) + i,
                                              0))
                   for _ in range(NCOL)] +
                  [pl.BlockSpec((RB, dim), lambda i, r: (i, 0))],
        out_shape=[jax.ShapeDtypeStruct((RN, W), _f32)] * NCOL +
                  [jax.ShapeDtypeStruct((n, dim), _f32)],
        name="gin_bn_mean",
    )(y2, s2, q2, g2, bb2)


def _readout(x, kf, ms, wcat, bsum):
    n, fin = x.shape
    nb = n // RB
    c = wcat.shape[1]
    kdim = wcat.shape[0]

    def body(x_ref, kf_ref, m1, m2, m3, m4, w_ref, b_ref, o_ref):
        m0 = x_ref[...] * kf_ref[...]
        mcat = jnp.concatenate(
            [m0, m1[...], m2[...], m3[...], m4[...]], axis=1)
        logits = lax.dot_general(mcat, w_ref[...], (((1,), (0,)), ((), ())),
                                 preferred_element_type=_f32,
                                 precision=lax.Precision.HIGHEST) + b_ref[...]
        mx = jnp.max(logits, axis=1, keepdims=True)
        sh = logits - mx
        lse = jnp.log(jnp.sum(jnp.exp(sh), axis=1, keepdims=True))
        o_ref[...] = sh - lse

    dim = ms[0].shape[1]
    return pl.pallas_call(
        body,
        grid=(nb,),
        in_specs=[
            pl.BlockSpec((RB, fin), lambda i: (i, 0)),
            pl.BlockSpec((RB, 1), lambda i: (i, 0)),
        ] + [pl.BlockSpec((RB, dim), lambda i: (i, 0)) for _ in range(4)] + [
            pl.BlockSpec((kdim, c), lambda i: (0, 0)),
            pl.BlockSpec((1, c), lambda i: (0, 0)),
        ],
        out_specs=pl.BlockSpec((RB, c), lambda i: (i, 0)),
        out_shape=jax.ShapeDtypeStruct((n, c), _f32),
        name="gin_readout",
    )(x, kf, *ms, wcat, bsum)


def kernel(x, edge_index, batch, params):
    convs, bns, fcs = params
    n, fin = x.shape
    R = NUM_RUNS
    RN = R * n
    num_layers = len(convs)

    drop = jax.random.bernoulli(jax.random.key(42), P_DROP, (R, n))
    keep = (1.0 - drop.astype(_f32)).reshape(R, n, 1)
    kf = keep.mean(axis=0)

    src = edge_index[0]
    dst = edge_index[1]
    offset = jnp.max(edge_index) + 1
    run_off = (jnp.arange(R, dtype=edge_index.dtype)[:, None] * offset)
    rsrc = (src[None, :] + run_off).reshape(-1)
    rdst = (dst[None, :] + run_off).reshape(-1)
    RE = rsrc.shape[0]
    gran = NPROD * CHB
    RE_pad = ((RE + gran - 1) // gran) * gran
    if RE_pad != RE:
        pad = RE_pad - RE
        rsrc = jnp.concatenate([rsrc, jnp.zeros((pad,), rsrc.dtype)])
        rdst = jnp.concatenate([rdst, jnp.full((pad,), RN, rdst.dtype)])

    dsrc = jnp.zeros((CB,), jnp.int32)
    ddst = jnp.full((CB,), RN, jnp.int32)
    bsrc, bdst, cnts = _make_bucket(RN, RE_pad)(rsrc, rdst, dsrc, ddst)
    cap = _bucket_cap(RE_pad)
    bsrc = bsrc.reshape(NBKT, NPROD, cap // EB, EB)
    bdst = bdst.reshape(NBKT, NPROD, cap // EB, EB)

    dump_rows = 96
    nz = (RN // NBKT + dump_rows + NTEC * 8 - 1) // (NTEC * 8) * 8
    zcache = {}

    hs = _drop_expand(x, keep, fin // NCOL)

    ms = []
    for i in range(num_layers):
        w1, b1, g1, bb1, w2, b2 = convs[i]
        g, b = bns[i]
        W = hs[0].shape[1]
        if W not in zcache:
            zcache[W] = jnp.zeros((nz, W), _f32)
        seg = _make_segment_sum(RN, W, dump_rows)
        aggs = seg(bsrc, bdst, cnts, zcache[W], *hs)
        y1, s1, q1 = _mm1_stats(hs, aggs, w1, b1.reshape(1, -1))
        y2, s2, q2 = _bn_relu_mm2(y1, s1, q1, g1.reshape(1, -1),
                                  bb1.reshape(1, -1), w2, b2.reshape(1, -1))
        *hs, m = _bn_relu_mean(y2, s2, q2, g.reshape(1, -1),
                               b.reshape(1, -1), n)
        ms.append(m)

    wcat = jnp.concatenate([w for (w, _) in fcs], axis=0)
    bsum = sum(bb for (_, bb) in fcs).reshape(1, -1)
    return _readout(x, kf, ms, wcat, bsum)
